# Initial kernel scaffold; baseline (speedup 1.0000x reference)
import jax
import jax.numpy as jnp
from jax.experimental import pallas as pl


def kernel(locs, data, density, weight, bias):
    raise NotImplementedError("write your pallas kernel here")


# speedup vs baseline: 1.6418x; 1.6418x over previous
import functools

import jax
import jax.numpy as jnp
import numpy as np
from jax.experimental import pallas as pl

_NDIM = 2
_KS = 3
_DIL = 0.05
_RADIUS = 0.1
_CIN = 32
_COUT = 32
_B = 2
_N = 2048
_NCELLS = _KS * _KS
_IT = 256


def _offset_list():
    c = (_KS - 1) / 2.0
    offs = []
    for a in range(_KS):
        for b in range(_KS):
            offs.append(((a - c) * _DIL, (b - c) * _DIL))
    return offs


def _spline_w(q):
    sigma = 40.0 / (7.0 * np.pi * _RADIUS * _RADIUS)
    w = jnp.where(q <= 0.5, 6.0 * (q ** 3 - q ** 2) + 1.0,
                  2.0 * jnp.clip(1.0 - q, 0.0, None) ** 3)
    return sigma * jnp.where(q <= 1.0, w, 0.0)


def _body(locs_ref, data_ref, density_ref, weight_ref, bias_ref, out_ref):
    it = pl.program_id(1)
    xs = locs_ref[0, :, 0]
    ys = locs_ref[0, :, 1]
    invm = locs_ref[0, :, 2]
    den = density_ref[0, 0, :]
    wd = data_ref[0] * (1.0 / (invm * den))[:, None]

    qx = locs_ref[0, pl.ds(it * _IT, _IT), 0]
    qy = locs_ref[0, pl.ds(it * _IT, _IT), 1]

    dx = qx[:, None] - xs[None, :]
    dy = qy[:, None] - ys[None, :]
    acc = jnp.zeros((_IT, _COUT), dtype=jnp.float32)
    for c, (ox, oy) in enumerate(_offset_list()):
        dxo = dx + ox
        dyo = dy + oy
        d2 = dxo * dxo + dyo * dyo
        d = jnp.sqrt(d2 + 1e-12)
        w = _spline_w(d * (1.0 / _RADIUS))
        f = jnp.dot(w, wd, preferred_element_type=jnp.float32)
        acc = acc + jnp.dot(f, weight_ref[:, :, c].T,
                            preferred_element_type=jnp.float32)
    out_ref[0] = acc + bias_ref[:][None, :]


@functools.partial(jax.jit, static_argnames=("interpret",))
def kernel(locs, data, density, weight, bias, interpret=False):
    grid = (_B, _N // _IT)
    return pl.pallas_call(
        _body,
        grid=grid,
        in_specs=[
            pl.BlockSpec((1, _N, _NDIM + 1), lambda b, i: (b, 0, 0)),
            pl.BlockSpec((1, _N, _CIN), lambda b, i: (b, 0, 0)),
            pl.BlockSpec((1, 1, _N), lambda b, i: (b, 0, 0)),
            pl.BlockSpec((_COUT, _CIN, _NCELLS), lambda b, i: (0, 0, 0)),
            pl.BlockSpec((_COUT,), lambda b, i: (0,)),
        ],
        out_specs=pl.BlockSpec((1, _IT, _COUT), lambda b, i: (b, i, 0)),
        out_shape=jax.ShapeDtypeStruct((_B, _N, _COUT), jnp.float32),
        interpret=interpret,
    )(locs, data, density.reshape(_B, 1, _N), weight, bias)


# speedup vs baseline: 1.7769x; 1.0823x over previous
import functools

import jax
import jax.numpy as jnp
import numpy as np
from jax.experimental import pallas as pl

_NDIM = 2
_KS = 3
_DIL = 0.05
_RADIUS = 0.1
_CIN = 32
_COUT = 32
_B = 2
_N = 2048
_NCELLS = _KS * _KS
_IT = 256


def _offset_list():
    c = (_KS - 1) / 2.0
    offs = []
    for a in range(_KS):
        for b in range(_KS):
            offs.append(((a - c) * _DIL, (b - c) * _DIL))
    return offs


def _spline_w(q):
    sigma = 40.0 / (7.0 * np.pi * _RADIUS * _RADIUS)
    w = jnp.where(q <= 0.5, 6.0 * (q ** 3 - q ** 2) + 1.0,
                  2.0 * jnp.clip(1.0 - q, 0.0, None) ** 3)
    return sigma * jnp.where(q <= 1.0, w, 0.0)


def _body(locs_ref, data_ref, density_ref, weight_ref, bias_ref, out_ref):
    it = pl.program_id(1)
    xs = locs_ref[0, :, 0]
    ys = locs_ref[0, :, 1]
    invm = locs_ref[0, :, 2]
    den = density_ref[0, 0, :]
    wd = data_ref[0] * (1.0 / (invm * den))[:, None]

    qx = locs_ref[0, pl.ds(it * _IT, _IT), 0]
    qy = locs_ref[0, pl.ds(it * _IT, _IT), 1]

    dx = qx[:, None] - xs[None, :]
    dy = qy[:, None] - ys[None, :]
    wd16 = wd.astype(jnp.bfloat16)
    sigma = 40.0 / (7.0 * np.pi * _RADIUS * _RADIUS)
    acc = jnp.zeros((_IT, _COUT), dtype=jnp.float32)
    for c, (ox, oy) in enumerate(_offset_list()):
        dxo = dx + ox
        dyo = dy + oy
        d2 = dxo * dxo + dyo * dyo
        q = jnp.sqrt(d2 + 1e-12) * (1.0 / _RADIUS)
        a = jnp.maximum(1.0 - q, 0.0)
        b = jnp.maximum(0.5 - q, 0.0)
        a3 = a * a * a
        b3 = b * b * b
        w = (2.0 * sigma) * (a3 - 4.0 * b3)
        f = jnp.dot(w.astype(jnp.bfloat16), wd16,
                    preferred_element_type=jnp.float32)
        acc = acc + jnp.dot(f, weight_ref[:, :, c].T,
                            preferred_element_type=jnp.float32)
    out_ref[0] = acc + bias_ref[:][None, :]


@functools.partial(jax.jit, static_argnames=("interpret",))
def kernel(locs, data, density, weight, bias, interpret=False):
    grid = (_B, _N // _IT)
    return pl.pallas_call(
        _body,
        grid=grid,
        in_specs=[
            pl.BlockSpec((1, _N, _NDIM + 1), lambda b, i: (b, 0, 0)),
            pl.BlockSpec((1, _N, _CIN), lambda b, i: (b, 0, 0)),
            pl.BlockSpec((1, 1, _N), lambda b, i: (b, 0, 0)),
            pl.BlockSpec((_COUT, _CIN, _NCELLS), lambda b, i: (0, 0, 0)),
            pl.BlockSpec((_COUT,), lambda b, i: (0,)),
        ],
        out_specs=pl.BlockSpec((1, _IT, _COUT), lambda b, i: (b, i, 0)),
        out_shape=jax.ShapeDtypeStruct((_B, _N, _COUT), jnp.float32),
        interpret=interpret,
    )(locs, data, density.reshape(_B, 1, _N), weight, bias)
